# baseline (speedup 1.0000x reference)
import functools

import jax
import jax.numpy as jnp
from jax import lax
from jax.experimental import pallas as pl
from jax.experimental.pallas import tpu as pltpu
from jax.experimental.pallas import tpu_sc as plsc

N = 10000
E = 320000
D = 128

NW = 32
CPW = 192
ROUNDS = 2
NPAD = NW * CPW * ROUNDS
CH = 2000
NCH = E // CH
GRP = CH // 16
BATCH = 64
PCAP = 80

_NEG_INF = float("-inf")


def _sc_aggregate(node_fts, src, dst, fval):
    mesh = plsc.VectorSubcoreMesh(core_axis_name="c", subcore_axis_name="s")

    @functools.partial(
        pl.kernel,
        out_type=[
            jax.ShapeDtypeStruct((NPAD, D), jnp.float32),
            jax.ShapeDtypeStruct((NPAD, D), jnp.float32),
            jax.ShapeDtypeStruct((NPAD, D), jnp.float32),
        ],
        mesh=mesh,
        compiler_params=pltpu.CompilerParams(needs_layout_passes=False),
        scratch_types=[
            pltpu.VMEM((CPW, D), jnp.float32),
            pltpu.VMEM((CPW, D), jnp.float32),
            pltpu.VMEM((CPW, D), jnp.float32),
            pltpu.VMEM((CH,), jnp.int32),
            pltpu.VMEM((CH,), jnp.int32),
            pltpu.VMEM((CH,), jnp.int32),
            pltpu.VMEM((CH,), jnp.int32),
            pltpu.VMEM((CH,), jnp.float32),
            pltpu.VMEM((CH,), jnp.float32),
            pltpu.VMEM((PCAP,), jnp.int32),
            pltpu.VMEM((PCAP,), jnp.int32),
            pltpu.VMEM((PCAP,), jnp.float32),
            pltpu.VMEM((BATCH,), jnp.int32),
            pltpu.VMEM((BATCH, D), jnp.float32),
            pltpu.SemaphoreType.DMA,
            pltpu.SemaphoreType.DMA,
            pltpu.SemaphoreType.DMA,
        ],
    )
    def agg(node_hbm, src_hbm, dst_hbm, f_hbm, sum_out, max_out, dir_out,
            sumacc, maxacc, diracc, dstb0, dstb1, srcb0, srcb1, fb0, fb1,
            psrc, pdst, pf, gidx, rows, semc0, semc1, semg):
        cid = lax.axis_index("c")
        sid = lax.axis_index("s")
        wid = cid * 16 + sid
        dstb = (dstb0, dstb1)
        srcb = (srcb0, srcb1)
        fb = (fb0, fb1)
        semc = (semc0, semc1)

        zero16 = jnp.zeros((16,), jnp.float32)
        neg16 = jnp.full((16,), _NEG_INF, jnp.float32)
        one16 = jnp.ones((16,), jnp.int32)
        zero16i = jnp.zeros((16,), jnp.int32)

        def start_chunk(ch, b):
            off = ch * CH
            pltpu.async_copy(dst_hbm.at[pl.ds(off, CH)], dstb[b], semc[b])
            pltpu.async_copy(src_hbm.at[pl.ds(off, CH)], srcb[b], semc[b])
            pltpu.async_copy(f_hbm.at[pl.ds(off, CH)], fb[b], semc[b])

        def wait_chunk(ch, b):
            off = ch * CH
            pltpu.make_async_copy(
                dst_hbm.at[pl.ds(off, CH)], dstb[b], semc[b]).wait()
            pltpu.make_async_copy(
                src_hbm.at[pl.ds(off, CH)], srcb[b], semc[b]).wait()
            pltpu.make_async_copy(
                f_hbm.at[pl.ds(off, CH)], fb[b], semc[b]).wait()

        def do_round(rnd):
            gbase = (rnd * NW + wid) * CPW

            def init_acc(i, carry):
                for k in range(D // 16):
                    sl = pl.ds(k * 16, 16)
                    sumacc[i, sl] = zero16
                    maxacc[i, sl] = neg16
                    diracc[i, sl] = zero16
                return carry
            lax.fori_loop(0, CPW, init_acc, 0)

            def flush(nvalid):
                for q in range(BATCH // 16):
                    gidx[pl.ds(q * 16, 16)] = psrc[pl.ds(q * 16, 16)]
                pltpu.async_copy(node_hbm.at[gidx], rows, semg).wait()

                def edge_body(i, carry):
                    c = pdst[pl.ds(i, 16)][0]
                    f = pf[pl.ds(i, 16)][0]
                    for k in range(D // 16):
                        sl = pl.ds(k * 16, 16)
                        r = rows[i, sl]
                        maxacc[c, sl] = jnp.maximum(maxacc[c, sl], r)
                        sumacc[c, sl] = sumacc[c, sl] + r
                        diracc[c, sl] = diracc[c, sl] + r * f
                    return carry
                lax.fori_loop(0, nvalid, edge_body, 0)

            def group_body(b, g, cnt):
                dvec = dstb[b][pl.ds(g * 16, 16)]
                loc = dvec - gbase
                mask = (loc >= 0) & (loc < CPW)
                pop = plsc.all_reduce_population_count(mask)[0]

                @pl.when(pop > 0)
                def _():
                    svec = srcb[b][pl.ds(g * 16, 16)]
                    fvec = fb[b][pl.ds(g * 16, 16)]
                    plsc.store_compressed(psrc.at[pl.ds(cnt, 16)], svec,
                                          mask=mask)
                    plsc.store_compressed(pdst.at[pl.ds(cnt, 16)], loc,
                                          mask=mask)
                    plsc.store_compressed(pf.at[pl.ds(cnt, 16)], fvec,
                                          mask=mask)
                cnt = cnt + pop

                @pl.when(cnt >= BATCH)
                def _():
                    flush(BATCH)
                    for name in (psrc, pdst, pf):
                        rem = name[pl.ds(BATCH, 16)]
                        name[pl.ds(0, 16)] = rem
                return jnp.where(cnt >= BATCH, cnt - BATCH, cnt)

            start_chunk(0, 0)

            def pair_body(j, cnt):
                ch0 = 2 * j
                start_chunk(ch0 + 1, 1)
                wait_chunk(ch0, 0)
                cnt = lax.fori_loop(
                    0, GRP, lambda g, c: group_body(0, g, c), cnt)

                @pl.when(j < NCH // 2 - 1)
                def _():
                    start_chunk(ch0 + 2, 0)
                wait_chunk(ch0 + 1, 1)
                cnt = lax.fori_loop(
                    0, GRP, lambda g, c: group_body(1, g, c), cnt)
                return cnt

            cnt = lax.fori_loop(0, NCH // 2, pair_body, 0)

            iota16 = lax.iota(jnp.int32, 16)
            for q in range(BATCH // 16):
                lane = iota16 + q * 16
                pad = lane >= cnt
                sv = psrc[pl.ds(q * 16, 16)]
                psrc[pl.ds(q * 16, 16)] = jnp.where(pad, zero16i, sv)
            flush(cnt)

            pltpu.sync_copy(sumacc, sum_out.at[pl.ds(gbase, CPW)])
            pltpu.sync_copy(maxacc, max_out.at[pl.ds(gbase, CPW)])
            pltpu.sync_copy(diracc, dir_out.at[pl.ds(gbase, CPW)])

        for rnd in range(ROUNDS):
            do_round(rnd)

    return agg(node_fts, src, dst, fval)


_ROWS = 400


def _post_body(node_ref, sum_ref, max_ref, dir_ref, deg_ref, fdig_ref,
               norm_ref, w_ref, b_ref, out_ref):
    node = node_ref[...]
    mean = sum_ref[...] / jnp.maximum(deg_ref[...], 1.0)
    mx = max_ref[...]
    mx = jnp.where(mx == -jnp.inf, 0.0, mx)
    dirv = dir_ref[...] - fdig_ref[...] * node
    w = w_ref[...]
    acc = (jnp.dot(node, w[0:128], preferred_element_type=jnp.float32)
           + jnp.dot(mean, w[128:256], preferred_element_type=jnp.float32)
           + jnp.dot(mx, w[256:384], preferred_element_type=jnp.float32)
           + jnp.dot(dirv, w[384:512], preferred_element_type=jnp.float32)
           + b_ref[...])
    out_ref[...] = node + norm_ref[...] * acc


def _post(node_fts, s, m, dir_sum, deg, fdig, norm_n, W_post, b_post):
    grid = (N // _ROWS,)
    row_spec = pl.BlockSpec((_ROWS, D), lambda i: (i, 0))
    col1_spec = pl.BlockSpec((_ROWS, 1), lambda i: (i, 0))
    return pl.pallas_call(
        _post_body,
        grid=grid,
        in_specs=[
            row_spec, row_spec, row_spec, row_spec,
            col1_spec, col1_spec, col1_spec,
            pl.BlockSpec((4 * D, D), lambda i: (0, 0)),
            pl.BlockSpec((1, D), lambda i: (0, 0)),
        ],
        out_specs=row_spec,
        out_shape=jax.ShapeDtypeStruct((N, D), jnp.float32),
    )(node_fts, s, m, dir_sum, deg, fdig, norm_n, W_post,
      b_post.reshape(1, D))


def kernel(node_fts, edge_fts, edge_index, F_norm_edge, F_dig, node_deg_vec,
           node_deg_mat, lap_mat, k_eig_val, k_eig_vec, num_nodes, norm_n,
           batch_idx, W_post, b_post):
    src = edge_index[0]
    dst = edge_index[1]
    fval = F_norm_edge.reshape(E)
    s_pad, m_pad, d_pad = _sc_aggregate(node_fts, src, dst, fval)
    return _post(node_fts, s_pad[:N], m_pad[:N], d_pad[:N], node_deg_vec,
                 F_dig, norm_n, W_post, b_post)


# speedup vs baseline: 1.3129x; 1.3129x over previous
import functools

import jax
import jax.numpy as jnp
from jax import lax
from jax.experimental import pallas as pl
from jax.experimental.pallas import tpu as pltpu
from jax.experimental.pallas import tpu_sc as plsc

N = 10000
E = 320000
D = 128

NW = 32
CPW = 192
ROUNDS = 2
NPAD = NW * CPW * ROUNDS
CH = 1280
NCH = E // CH
GRP = CH // 16
UNROLL = 8
BATCH = 64
PCAP = 224

_NEG_INF = float("-inf")


def _sc_aggregate(node_fts, src, dst, fval):
    mesh = plsc.VectorSubcoreMesh(core_axis_name="c", subcore_axis_name="s")

    @functools.partial(
        pl.kernel,
        out_type=[
            jax.ShapeDtypeStruct((NPAD, D), jnp.float32),
            jax.ShapeDtypeStruct((NPAD, D), jnp.float32),
            jax.ShapeDtypeStruct((NPAD, D), jnp.float32),
        ],
        mesh=mesh,
        compiler_params=pltpu.CompilerParams(needs_layout_passes=False),
        scratch_types=[
            pltpu.VMEM((CPW, D), jnp.float32),
            pltpu.VMEM((CPW, D), jnp.float32),
            pltpu.VMEM((CPW, D), jnp.float32),
            pltpu.VMEM((CH,), jnp.int32),
            pltpu.VMEM((CH,), jnp.int32),
            pltpu.VMEM((CH,), jnp.int32),
            pltpu.VMEM((CH,), jnp.int32),
            pltpu.VMEM((CH,), jnp.float32),
            pltpu.VMEM((CH,), jnp.float32),
            pltpu.VMEM((PCAP,), jnp.int32),
            pltpu.VMEM((PCAP,), jnp.int32),
            pltpu.VMEM((PCAP,), jnp.float32),
            pltpu.VMEM((BATCH,), jnp.int32),
            pltpu.VMEM((BATCH, D), jnp.float32),
            pltpu.SemaphoreType.DMA,
            pltpu.SemaphoreType.DMA,
            pltpu.SemaphoreType.DMA,
        ],
    )
    def agg(node_hbm, src_hbm, dst_hbm, f_hbm, sum_out, max_out, dir_out,
            sumacc, maxacc, diracc, dstb0, dstb1, srcb0, srcb1, fb0, fb1,
            psrc, pdst, pf, gidx, rows, semc0, semc1, semg):
        cid = lax.axis_index("c")
        sid = lax.axis_index("s")
        wid = cid * 16 + sid
        dstb = (dstb0, dstb1)
        srcb = (srcb0, srcb1)
        fb = (fb0, fb1)
        semc = (semc0, semc1)

        zero16 = jnp.zeros((16,), jnp.float32)
        neg16 = jnp.full((16,), _NEG_INF, jnp.float32)
        one16 = jnp.ones((16,), jnp.int32)
        zero16i = jnp.zeros((16,), jnp.int32)

        def start_chunk(ch, b):
            off = ch * CH
            pltpu.async_copy(dst_hbm.at[pl.ds(off, CH)], dstb[b], semc[b])
            pltpu.async_copy(src_hbm.at[pl.ds(off, CH)], srcb[b], semc[b])
            pltpu.async_copy(f_hbm.at[pl.ds(off, CH)], fb[b], semc[b])

        def wait_chunk(ch, b):
            off = ch * CH
            pltpu.make_async_copy(
                dst_hbm.at[pl.ds(off, CH)], dstb[b], semc[b]).wait()
            pltpu.make_async_copy(
                src_hbm.at[pl.ds(off, CH)], srcb[b], semc[b]).wait()
            pltpu.make_async_copy(
                f_hbm.at[pl.ds(off, CH)], fb[b], semc[b]).wait()

        def do_round(rnd):
            gbase = (rnd * NW + wid) * CPW

            def init_acc(i, carry):
                for k in range(D // 16):
                    sl = pl.ds(k * 16, 16)
                    sumacc[i, sl] = zero16
                    maxacc[i, sl] = neg16
                    diracc[i, sl] = zero16
                return carry
            lax.fori_loop(0, CPW, init_acc, 0)

            def flush(nvalid):
                for q in range(BATCH // 16):
                    gidx[pl.ds(q * 16, 16)] = psrc[pl.ds(q * 16, 16)]
                pltpu.async_copy(node_hbm.at[gidx], rows, semg).wait()

                def edge_body(i, carry):
                    c = pdst[pl.ds(i, 16)][0]
                    f = pf[pl.ds(i, 16)][0]
                    for k in range(D // 16):
                        sl = pl.ds(k * 16, 16)
                        r = rows[i, sl]
                        maxacc[c, sl] = jnp.maximum(maxacc[c, sl], r)
                        sumacc[c, sl] = sumacc[c, sl] + r
                        diracc[c, sl] = diracc[c, sl] + r * f
                    return carry
                lax.fori_loop(0, nvalid, edge_body, 0)

            def block_body(b, g8, cntv):
                prefix = cntv
                for u in range(UNROLL):
                    off = (g8 * UNROLL + u) * 16
                    dvec = dstb[b][pl.ds(off, 16)]
                    loc = dvec - gbase
                    mask = (loc >= 0) & (loc < CPW)
                    mi = jnp.where(mask, one16, zero16i)
                    csum = plsc.cumsum(mi)
                    pos = prefix + csum - 1
                    svec = srcb[b][pl.ds(off, 16)]
                    fvec = fb[b][pl.ds(off, 16)]
                    plsc.store_scatter(psrc, [pos], svec, mask=mask)
                    plsc.store_scatter(pdst, [pos], loc, mask=mask)
                    plsc.store_scatter(pf, [pos], fvec, mask=mask)
                    prefix = prefix + plsc.all_reduce_population_count(mask)
                cnt = prefix[0]

                @pl.when(cnt >= BATCH)
                def _():
                    flush(BATCH)
                    for name in (psrc, pdst, pf):
                        for q in range(8):
                            rem = name[pl.ds(BATCH + q * 16, 16)]
                            name[pl.ds(q * 16, 16)] = rem

                @pl.when(cnt >= 2 * BATCH)
                def _():
                    flush(BATCH)
                    for name in (psrc, pdst, pf):
                        for q in range(8):
                            rem = name[pl.ds(BATCH + q * 16, 16)]
                            name[pl.ds(q * 16, 16)] = rem

                batch16 = jnp.full((16,), BATCH, jnp.int32)
                cntv = prefix
                cntv = jnp.where(cnt >= BATCH, cntv - batch16, cntv)
                cntv = jnp.where(cnt >= 2 * BATCH, cntv - batch16, cntv)
                return cntv

            start_chunk(0, 0)

            def pair_body(j, cntv):
                ch0 = 2 * j
                start_chunk(ch0 + 1, 1)
                wait_chunk(ch0, 0)
                cntv = lax.fori_loop(
                    0, GRP // UNROLL, lambda g, c: block_body(0, g, c), cntv)

                @pl.when(j < NCH // 2 - 1)
                def _():
                    start_chunk(ch0 + 2, 0)
                wait_chunk(ch0 + 1, 1)
                cntv = lax.fori_loop(
                    0, GRP // UNROLL, lambda g, c: block_body(1, g, c), cntv)
                return cntv

            cntv = lax.fori_loop(0, NCH // 2, pair_body, zero16i)
            cnt = cntv[0]

            iota16 = lax.iota(jnp.int32, 16)
            for q in range(BATCH // 16):
                lane = iota16 + q * 16
                pad = lane >= cnt
                sv = psrc[pl.ds(q * 16, 16)]
                psrc[pl.ds(q * 16, 16)] = jnp.where(pad, zero16i, sv)
            flush(cnt)

            pltpu.sync_copy(sumacc, sum_out.at[pl.ds(gbase, CPW)])
            pltpu.sync_copy(maxacc, max_out.at[pl.ds(gbase, CPW)])
            pltpu.sync_copy(diracc, dir_out.at[pl.ds(gbase, CPW)])

        for rnd in range(ROUNDS):
            do_round(rnd)

    return agg(node_fts, src, dst, fval)


_ROWS = 400


def _post_body(node_ref, sum_ref, max_ref, dir_ref, deg_ref, fdig_ref,
               norm_ref, w_ref, b_ref, out_ref):
    node = node_ref[...]
    mean = sum_ref[...] / jnp.maximum(deg_ref[...], 1.0)
    mx = max_ref[...]
    mx = jnp.where(mx == -jnp.inf, 0.0, mx)
    dirv = dir_ref[...] - fdig_ref[...] * node
    w = w_ref[...]
    acc = (jnp.dot(node, w[0:128], preferred_element_type=jnp.float32)
           + jnp.dot(mean, w[128:256], preferred_element_type=jnp.float32)
           + jnp.dot(mx, w[256:384], preferred_element_type=jnp.float32)
           + jnp.dot(dirv, w[384:512], preferred_element_type=jnp.float32)
           + b_ref[...])
    out_ref[...] = node + norm_ref[...] * acc


def _post(node_fts, s, m, dir_sum, deg, fdig, norm_n, W_post, b_post):
    grid = (N // _ROWS,)
    row_spec = pl.BlockSpec((_ROWS, D), lambda i: (i, 0))
    col1_spec = pl.BlockSpec((_ROWS, 1), lambda i: (i, 0))
    return pl.pallas_call(
        _post_body,
        grid=grid,
        in_specs=[
            row_spec, row_spec, row_spec, row_spec,
            col1_spec, col1_spec, col1_spec,
            pl.BlockSpec((4 * D, D), lambda i: (0, 0)),
            pl.BlockSpec((1, D), lambda i: (0, 0)),
        ],
        out_specs=row_spec,
        out_shape=jax.ShapeDtypeStruct((N, D), jnp.float32),
    )(node_fts, s, m, dir_sum, deg, fdig, norm_n, W_post,
      b_post.reshape(1, D))


def kernel(node_fts, edge_fts, edge_index, F_norm_edge, F_dig, node_deg_vec,
           node_deg_mat, lap_mat, k_eig_val, k_eig_vec, num_nodes, norm_n,
           batch_idx, W_post, b_post):
    src = edge_index[0]
    dst = edge_index[1]
    fval = F_norm_edge.reshape(E)
    s_pad, m_pad, d_pad = _sc_aggregate(node_fts, src, dst, fval)
    return _post(node_fts, s_pad[:N], m_pad[:N], d_pad[:N], node_deg_vec,
                 F_dig, norm_n, W_post, b_post)
